# baseline (speedup 1.0000x reference)
import jax
import jax.numpy as jnp
from jax import lax
from jax.experimental import pallas as pl
from jax.experimental.pallas import tpu as pltpu
from jax.experimental.pallas import tpu_sc as plsc

B, L, D = 4096, 200, 128
N = B * L
NC, NS = 2, 16
NW = NC * NS
PER_W = N // NW
AV, NV = 121, 17
CH = 128
NCHUNK = PER_W // CH
NBUF = 4


def _mesh():
    return plsc.VectorSubcoreMesh(core_axis_name="c", subcore_axis_name="s")


def _wid():
    return lax.axis_index("s") * NC + lax.axis_index("c")


def _zero_body(at_hbm, nt_hbm, atz_hbm, ntz_hbm, at_v, nt_v):
    @pl.when(_wid() == 0)
    def _():
        pltpu.sync_copy(at_hbm, at_v)
        pltpu.sync_copy(nt_hbm, nt_v)
        zeros_f = jnp.zeros((16,), jnp.float32)
        for k in range(8):
            at_v[pl.ds(k * 16, 16)] = zeros_f
            nt_v[pl.ds(k * 16, 16)] = zeros_f
        pltpu.sync_copy(at_v, atz_hbm)
        pltpu.sync_copy(nt_v, ntz_hbm)


def _gather_body(atoms_hbm, neigh_hbm, atz_hbm, ntz_hbm, out_hbm,
                 ia_v, in_v, r0, r1, r2, r3, at_sp, nt_sp,
                 a0, a1, a2, a3, n0, n1, n2, n3, o0, o1, o2, o3):
    rows = (r0, r1, r2, r3)
    asem = (a0, a1, a2, a3)
    nsem = (n0, n1, n2, n3)
    osem = (o0, o1, o2, o3)

    w = _wid()
    base = w * PER_W

    pltpu.sync_copy(atz_hbm, r0.at[pl.ds(0, AV)])
    pltpu.sync_copy(r0.at[pl.ds(0, AV)], at_sp)
    pltpu.sync_copy(ntz_hbm, r1.at[pl.ds(0, NV)])
    pltpu.sync_copy(r1.at[pl.ds(0, NV)], nt_sp)

    pltpu.sync_copy(atoms_hbm.at[pl.ds(base, PER_W)], ia_v)
    pltpu.sync_copy(neigh_hbm.at[pl.ds(base, PER_W)], in_v)

    def start_g1(ci, b):
        idxs = ia_v.at[pl.ds(ci * CH, CH)]
        pltpu.async_copy(at_sp.at[idxs], rows[b], asem[b])

    def wait_g1(b):
        pltpu.make_async_copy(at_sp.at[pl.ds(0, CH)], rows[b],
                              asem[b]).wait()

    def start_g2(ci, b, parity):
        idxs = in_v.at[pl.ds(ci * CH, CH)]
        src = nt_sp if parity == 0 else ntz_hbm
        pltpu.async_copy(src.at[idxs], rows[b], nsem[b], add=True)

    def wait_g2(b):
        pltpu.make_async_copy(nt_sp.at[pl.ds(0, CH)], rows[b],
                              nsem[b]).wait()

    def start_out(ci, b):
        dst = out_hbm.at[pl.ds(base + ci * CH, CH)]
        pltpu.async_copy(rows[b], dst, osem[b])

    def wait_out(b):
        pltpu.make_async_copy(rows[b], out_hbm.at[pl.ds(0, CH)],
                              osem[b]).wait()

    start_g1(0, 0)
    start_g1(1, 1)
    wait_g1(0)
    start_g2(0, 0, 0)

    @pl.loop(0, NCHUNK // NBUF)
    def _quad(k):
        for j in range(NBUF):
            ci = k * NBUF + j

            @pl.when(ci + 2 < NCHUNK)
            def _():
                b2 = (j + 2) % NBUF

                @pl.when(ci >= 2)
                def _():
                    wait_out(b2)

                start_g1(ci + 2, b2)

            @pl.when(ci + 1 < NCHUNK)
            def _():
                b1 = (j + 1) % NBUF
                wait_g1(b1)
                start_g2(ci + 1, b1, (j + 1) % 2)

            wait_g2(j)
            start_out(ci, j)

    for b in range(NBUF):
        wait_out(b)


@jax.jit
def _run(atoms_flat, neigh_flat, at_flat, nt_flat):
    zero = pl.kernel(
        _zero_body,
        out_type=(jax.ShapeDtypeStruct((AV * D,), jnp.float32),
                  jax.ShapeDtypeStruct((NV * D,), jnp.float32)),
        mesh=_mesh(),
        compiler_params=pltpu.CompilerParams(needs_layout_passes=False),
        scratch_types=[
            pltpu.VMEM((AV * D,), jnp.float32),
            pltpu.VMEM((NV * D,), jnp.float32),
        ],
    )
    atz, ntz = zero(at_flat, nt_flat)

    gather = pl.kernel(
        _gather_body,
        out_type=jax.ShapeDtypeStruct((N, D), jnp.float32),
        mesh=_mesh(),
        compiler_params=pltpu.CompilerParams(needs_layout_passes=False),
        scratch_types=[
            pltpu.VMEM((PER_W,), jnp.int32),
            pltpu.VMEM((PER_W,), jnp.int32),
            pltpu.VMEM((CH, D), jnp.float32),
            pltpu.VMEM((CH, D), jnp.float32),
            pltpu.VMEM((CH, D), jnp.float32),
            pltpu.VMEM((CH, D), jnp.float32),
            pltpu.VMEM_SHARED((AV, D), jnp.float32),
            pltpu.VMEM_SHARED((NV, D), jnp.float32),
            pltpu.SemaphoreType.DMA,
            pltpu.SemaphoreType.DMA,
            pltpu.SemaphoreType.DMA,
            pltpu.SemaphoreType.DMA,
            pltpu.SemaphoreType.DMA,
            pltpu.SemaphoreType.DMA,
            pltpu.SemaphoreType.DMA,
            pltpu.SemaphoreType.DMA,
            pltpu.SemaphoreType.DMA,
            pltpu.SemaphoreType.DMA,
            pltpu.SemaphoreType.DMA,
            pltpu.SemaphoreType.DMA,
        ],
    )
    return gather(atoms_flat, neigh_flat, atz.reshape(AV, D),
                  ntz.reshape(NV, D))


def kernel(atoms, neighbors, atoms_table, neighbors_table):
    out = _run(atoms.reshape(N), neighbors.reshape(N),
               atoms_table.reshape(AV * D), neighbors_table.reshape(NV * D))
    return out.reshape(B, L, D)


# speedup vs baseline: 3.8996x; 3.8996x over previous
import jax
import jax.numpy as jnp
from jax import lax
from jax.experimental import pallas as pl
from jax.experimental.pallas import tpu as pltpu
from jax.experimental.pallas import tpu_sc as plsc

B, L, D = 4096, 200, 128
N = B * L
NC, NS = 2, 16
NW = NC * NS
PER_W = N // NW
AV, NV = 121, 17
NCOMB = AV * NV
ROWS_W = 65
NCOMB_PAD = ROWS_W * NW
CH = 128
NCHUNK = PER_W // CH
NBUF = 4


def _mesh():
    return plsc.VectorSubcoreMesh(core_axis_name="c", subcore_axis_name="s")


def _wid():
    return lax.axis_index("s") * NC + lax.axis_index("c")


def _build_body(at_hbm, nt_hbm, comb_hbm, at_v, nt_v, buf):
    w = _wid()
    start = w * ROWS_W

    pltpu.sync_copy(at_hbm, at_v)
    pltpu.sync_copy(nt_hbm, nt_v)

    zeros_f = jnp.zeros((16,), jnp.float32)
    for k in range(8):
        at_v[pl.ds(k * 16, 16)] = zeros_f
        nt_v[pl.ds(k * 16, 16)] = zeros_f

    @pl.loop(0, ROWS_W)
    def _row(ri):
        r = start + ri

        @pl.when(r < NCOMB)
        def _():
            a = r // NV
            n = r - a * NV
            for k in range(8):
                va = at_v[pl.ds(a * D + k * 16, 16)]
                vn = nt_v[pl.ds(n * D + k * 16, 16)]
                buf[pl.ds(ri * D + k * 16, 16)] = va + vn

    pltpu.sync_copy(buf, comb_hbm.at[pl.ds(start * D, ROWS_W * D)])


def _gather_body(atoms_hbm, neigh_hbm, comb_hbm, out_hbm,
                 ia_v, in_v, r0, r1, r2, r3, g0, g1, g2, g3, o0, o1, o2, o3):
    rows = (r0, r1, r2, r3)
    gsem = (g0, g1, g2, g3)
    osem = (o0, o1, o2, o3)

    w = _wid()
    base = w * PER_W

    pltpu.sync_copy(atoms_hbm.at[pl.ds(base, PER_W)], ia_v)
    pltpu.sync_copy(neigh_hbm.at[pl.ds(base, PER_W)], in_v)

    @pl.loop(0, PER_W // 16)
    def _fold(i):
        off = i * 16
        ia_v[pl.ds(off, 16)] = ia_v[pl.ds(off, 16)] * NV + in_v[pl.ds(off, 16)]

    def start_gather(ci, b):
        idxs = ia_v.at[pl.ds(ci * CH, CH)]
        pltpu.async_copy(comb_hbm.at[idxs], rows[b], gsem[b])

    def wait_gather(b):
        pltpu.make_async_copy(comb_hbm.at[pl.ds(0, CH)], rows[b],
                              gsem[b]).wait()

    def start_out(ci, b):
        dst = out_hbm.at[pl.ds(base + ci * CH, CH)]
        pltpu.async_copy(rows[b], dst, osem[b])

    def wait_out(b):
        pltpu.make_async_copy(rows[b], out_hbm.at[pl.ds(0, CH)],
                              osem[b]).wait()

    start_gather(0, 0)
    start_gather(1, 1)

    @pl.loop(0, NCHUNK // NBUF)
    def _quad(k):
        for j in range(NBUF):
            ci = k * NBUF + j
            b = j
            b2 = (j + 2) % NBUF
            ci2 = ci + 2

            @pl.when(ci2 >= NBUF)
            def _():
                wait_out(b2)

            @pl.when(ci2 < NCHUNK)
            def _():
                start_gather(ci2, b2)

            wait_gather(b)
            start_out(ci, b)

    wait_out((NCHUNK - 2) % NBUF)
    wait_out((NCHUNK - 1) % NBUF)


@jax.jit
def _run(atoms_flat, neigh_flat, at_flat, nt_flat):
    build = pl.kernel(
        _build_body,
        out_type=jax.ShapeDtypeStruct((NCOMB_PAD * D,), jnp.float32),
        mesh=_mesh(),
        compiler_params=pltpu.CompilerParams(needs_layout_passes=False),
        scratch_types=[
            pltpu.VMEM((AV * D,), jnp.float32),
            pltpu.VMEM((NV * D,), jnp.float32),
            pltpu.VMEM((ROWS_W * D,), jnp.float32),
        ],
    )
    comb = build(at_flat, nt_flat).reshape(NCOMB_PAD, D)

    gather = pl.kernel(
        _gather_body,
        out_type=jax.ShapeDtypeStruct((N, D), jnp.float32),
        mesh=_mesh(),
        compiler_params=pltpu.CompilerParams(needs_layout_passes=False),
        scratch_types=[
            pltpu.VMEM((PER_W,), jnp.int32),
            pltpu.VMEM((PER_W,), jnp.int32),
            pltpu.VMEM((CH, D), jnp.float32),
            pltpu.VMEM((CH, D), jnp.float32),
            pltpu.VMEM((CH, D), jnp.float32),
            pltpu.VMEM((CH, D), jnp.float32),
            pltpu.SemaphoreType.DMA,
            pltpu.SemaphoreType.DMA,
            pltpu.SemaphoreType.DMA,
            pltpu.SemaphoreType.DMA,
            pltpu.SemaphoreType.DMA,
            pltpu.SemaphoreType.DMA,
            pltpu.SemaphoreType.DMA,
            pltpu.SemaphoreType.DMA,
        ],
    )
    return gather(atoms_flat, neigh_flat, comb)


def kernel(atoms, neighbors, atoms_table, neighbors_table):
    out = _run(atoms.reshape(N), neighbors.reshape(N),
               atoms_table.reshape(AV * D), neighbors_table.reshape(NV * D))
    return out.reshape(B, L, D)
